# baseline (speedup 1.0000x reference)
import functools

import jax
import jax.numpy as jnp
from jax import lax
from jax.experimental import pallas as pl
from jax.experimental.pallas import tpu as pltpu
from jax.experimental.pallas import tpu_sc as plsc

_INSTANCE_COL = 13
_L = 16
_NC = 2
_NS = 16
_NW = _NC * _NS


@functools.lru_cache(maxsize=None)
def _build(B, F):
    b_per_w = B // _NW
    chunk = b_per_w * F
    mesh = plsc.VectorSubcoreMesh(core_axis_name="c", subcore_axis_name="s")

    @functools.partial(
        pl.kernel,
        out_type=jax.ShapeDtypeStruct((B * F,), jnp.float32),
        mesh=mesh,
        compiler_params=pltpu.CompilerParams(needs_layout_passes=False),
        scratch_types=[
            pltpu.VMEM((chunk,), jnp.float32),
            pltpu.VMEM((b_per_w,), jnp.int32),
            pltpu.VMEM((b_per_w,), jnp.float32),
            pltpu.SemaphoreType.DMA,
        ],
    )
    def k(feats_hbm, table_hbm, out_hbm, feats_v, idx_v, vals_v, sem):
        wid = lax.axis_index("s") * _NC + lax.axis_index("c")
        base = wid * chunk
        pltpu.sync_copy(feats_hbm.at[pl.ds(base, chunk)], feats_v)
        for i in range(b_per_w // _L):
            offs = lax.iota(jnp.int32, _L) * jnp.int32(F) + jnp.int32(
                i * _L * F + _INSTANCE_COL
            )
            ids_f = plsc.load_gather(feats_v, [offs])
            idx_v[pl.ds(i * _L, _L)] = ids_f.astype(jnp.int32)
        pltpu.async_copy(table_hbm.at[idx_v], vals_v, sem).wait()
        for i in range(b_per_w // _L):
            offs = lax.iota(jnp.int32, _L) * jnp.int32(F) + jnp.int32(
                i * _L * F + _INSTANCE_COL
            )
            plsc.store_scatter(feats_v, [offs], vals_v[pl.ds(i * _L, _L)])
        pltpu.sync_copy(feats_v, out_hbm.at[pl.ds(base, chunk)])

    return k


def kernel(self_feats, emb_table):
    B, F = self_feats.shape
    out = _build(B, F)(self_feats.reshape(-1), emb_table.reshape(-1))
    return out.reshape(B, F)
